# baseline (speedup 1.0000x reference)
import jax
import jax.numpy as jnp
from jax.experimental import pallas as pl

_LANES = 128
_GROUP = 16
_GPL = _LANES // _GROUP


def _mask_body(x_ref, o_ref):
    x = x_ref[...]
    blk, rows, lanes = x.shape
    sq = x * x

    li = jax.lax.broadcasted_iota(jnp.int32, (lanes, _GPL), 0)
    gi = jax.lax.broadcasted_iota(jnp.int32, (lanes, _GPL), 1)
    gmat = (li // _GROUP == gi).astype(jnp.float32)
    s = jax.lax.dot_general(
        sq, gmat, (((2,), (0,)), ((), ())),
        preferred_element_type=jnp.float32,
        precision=jax.lax.Precision.HIGHEST,
    )

    m = jnp.max(jnp.max(s, axis=2, keepdims=True), axis=1, keepdims=True)

    ri = jax.lax.broadcasted_iota(jnp.int32, (blk, rows, _GPL), 1)
    ci = jax.lax.broadcasted_iota(jnp.int32, (blk, rows, _GPL), 2)
    flat = ri * _GPL + ci
    big = jnp.int32(1 << 30)
    idx = jnp.min(
        jnp.min(jnp.where(s == m, flat, big), axis=2), axis=1
    )

    r2 = jax.lax.broadcasted_iota(jnp.int32, (blk, rows, lanes), 1)
    l2 = jax.lax.broadcasted_iota(jnp.int32, (blk, rows, lanes), 2)
    cap = r2 * _GPL + l2 // _GROUP
    keep = cap == idx[:, None, None]
    o_ref[...] = jnp.where(keep, x, jnp.float32(0.0))


def kernel(inputs):
    b, c, d = inputs.shape
    flat = c * d
    rows = flat // _LANES
    blk = 64
    x = inputs.reshape(b, rows, _LANES)
    out = pl.pallas_call(
        _mask_body,
        grid=(b // blk,),
        in_specs=[pl.BlockSpec((blk, rows, _LANES), lambda i: (i, 0, 0))],
        out_specs=pl.BlockSpec((blk, rows, _LANES), lambda i: (i, 0, 0)),
        out_shape=jax.ShapeDtypeStruct((b, rows, _LANES), jnp.float32),
    )(x)
    return out.reshape(b, flat)


# speedup vs baseline: 1.0124x; 1.0124x over previous
import jax
import jax.numpy as jnp
from jax.experimental import pallas as pl

_LANES = 128
_GROUP = 16
_GPL = _LANES // _GROUP


def _mask_body(x_ref, o_ref):
    x = x_ref[...]
    blk, rows, lanes = x.shape
    sq = x * x

    li = jax.lax.broadcasted_iota(jnp.int32, (lanes, lanes), 0)
    co = jax.lax.broadcasted_iota(jnp.int32, (lanes, lanes), 1)
    gmat = (li // _GROUP == co // _GROUP).astype(jnp.float32)
    sg = jax.lax.dot_general(
        sq, gmat, (((2,), (0,)), ((), ())),
        preferred_element_type=jnp.float32,
        precision=jax.lax.Precision.HIGHEST,
    )

    m = jnp.max(jnp.max(sg, axis=2, keepdims=True), axis=1, keepdims=True)

    r2d = jax.lax.broadcasted_iota(jnp.int32, (rows, lanes), 0)
    l2d = jax.lax.broadcasted_iota(jnp.int32, (rows, lanes), 1)
    flat = (r2d * _GPL + l2d // _GROUP)[None]

    big = jnp.int32(1 << 30)
    wh = jnp.where(sg == m, flat, big)
    idx = jnp.min(jnp.min(wh, axis=2, keepdims=True), axis=1, keepdims=True)

    keep = flat == idx
    o_ref[...] = jnp.where(keep, x, jnp.float32(0.0))


def kernel(inputs):
    b, c, d = inputs.shape
    flat = c * d
    rows = flat // _LANES
    blk = 64
    x = inputs.reshape(b, rows, _LANES)
    out = pl.pallas_call(
        _mask_body,
        grid=(b // blk,),
        in_specs=[pl.BlockSpec((blk, rows, _LANES), lambda i: (i, 0, 0))],
        out_specs=pl.BlockSpec((blk, rows, _LANES), lambda i: (i, 0, 0)),
        out_shape=jax.ShapeDtypeStruct((b, rows, _LANES), jnp.float32),
    )(x)
    return out.reshape(b, flat)


# speedup vs baseline: 2.3767x; 2.3477x over previous
import jax
import jax.numpy as jnp
from jax.experimental import pallas as pl

_LANES = 128
_GROUP = 16
_GPL = _LANES // _GROUP


def _mask_body(x_ref, o_ref):
    x = x_ref[...]
    blk, flatdim = x.shape
    sq = x * x

    li = jax.lax.broadcasted_iota(jnp.int32, (_LANES, _LANES), 0)
    co = jax.lax.broadcasted_iota(jnp.int32, (_LANES, _LANES), 1)
    gmat = (li // _GROUP == co // _GROUP).astype(jnp.float32)
    parts = []
    for r in range(flatdim // _LANES):
        parts.append(
            jax.lax.dot_general(
                sq[:, r * _LANES:(r + 1) * _LANES], gmat,
                (((1,), (0,)), ((), ())),
                preferred_element_type=jnp.float32,
                precision=jax.lax.Precision.HIGHEST,
            )
        )
    sg = jnp.concatenate(parts, axis=1)

    m = jnp.max(sg, axis=1, keepdims=True)

    flat = jax.lax.broadcasted_iota(jnp.int32, (1, flatdim), 1) // _GROUP

    big = jnp.int32(1 << 30)
    wh = jnp.where(sg == m, flat, big)
    idx = jnp.min(wh, axis=1, keepdims=True)

    keep = flat == idx
    o_ref[...] = jnp.where(keep, x, jnp.float32(0.0))


def kernel(inputs):
    b, c, d = inputs.shape
    flat = c * d
    blk = 64
    x = inputs.reshape(b, flat)
    return pl.pallas_call(
        _mask_body,
        grid=(b // blk,),
        in_specs=[pl.BlockSpec((blk, flat), lambda i: (i, 0))],
        out_specs=pl.BlockSpec((blk, flat), lambda i: (i, 0)),
        out_shape=jax.ShapeDtypeStruct((b, flat), jnp.float32),
    )(x)
